# baseline (speedup 1.0000x reference)
import functools

import jax
import jax.numpy as jnp
from jax import lax
from jax.experimental import pallas as pl
from jax.experimental.pallas import tpu as pltpu
from jax.experimental.pallas import tpu_sc as plsc

B = 16
L = 8192
V = 4
PAD = 5000
LOUT = L + 2 * PAD
HALF = L // 2
ZROW = 5008
LANES = 16


def _zero_template(zpad_v):
    zeros16 = jnp.zeros((LANES,), jnp.float32)

    def body(i, c):
        zpad_v[pl.ds(i * LANES, LANES)] = zeros16
        return c

    lax.fori_loop(0, ZROW // LANES, body, 0)


def _compute_half(ids_v, mask_v, data_v):
    def body(i, c):
        base = i * (4 * LANES)
        for u in range(4):
            off = base + u * LANES
            ids16 = ids_v[pl.ds(off, LANES)]
            m16 = mask_v[pl.ds(off, LANES)]
            for v in range(V):
                data_v[pl.ds(v * HALF + off, LANES)] = jnp.where(
                    ids16 == v, m16, 0.0
                )
        return c

    lax.fori_loop(0, HALF // (4 * LANES), body, 0)


@functools.partial(
    pl.kernel,
    mesh=plsc.VectorSubcoreMesh(core_axis_name="c", subcore_axis_name="s"),
    out_type=jax.ShapeDtypeStruct((B * V * LOUT,), jnp.float32),
    scratch_types=[
        pltpu.VMEM((HALF,), jnp.int32),
        pltpu.VMEM((HALF,), jnp.float32),
        pltpu.VMEM((V * HALF,), jnp.float32),
        pltpu.VMEM((ZROW,), jnp.float32),
    ],
)
def _one_hot_pad_sc(ids_hbm, mask_hbm, out_hbm, ids_v, mask_v, data_v, zpad_v):
    info = plsc.get_sparse_core_info()
    wid = lax.axis_index("s") * info.num_cores + lax.axis_index("c")
    b = wid // 2
    h = wid % 2

    _zero_template(zpad_v)

    in_off = pl.multiple_of(b * L + h * HALF, HALF)
    pltpu.sync_copy(ids_hbm.at[pl.ds(in_off, HALF)], ids_v)
    pltpu.sync_copy(mask_hbm.at[pl.ds(in_off, HALF)], mask_v)
    _compute_half(ids_v, mask_v, data_v)

    for v in range(V):
        row_off = (b * V + v) * LOUT
        data_off = pl.multiple_of(row_off + PAD + h * HALF, 8)
        pltpu.sync_copy(
            data_v.at[pl.ds(v * HALF, HALF)], out_hbm.at[pl.ds(data_off, HALF)]
        )
        pad_off = pl.multiple_of(row_off + h * (PAD + L), 8)
        pltpu.sync_copy(
            zpad_v.at[pl.ds(0, PAD)], out_hbm.at[pl.ds(pad_off, PAD)]
        )


def kernel(input_ids, attention_mask):
    ids = input_ids.astype(jnp.int32).reshape(B * L)
    mask = attention_mask.astype(jnp.float32).reshape(B * L)
    out_flat = _one_hot_pad_sc(ids, mask)
    return out_flat.reshape(B, V, LOUT)


# speedup vs baseline: 1.0902x; 1.0902x over previous
import functools

import jax
import jax.numpy as jnp
from jax import lax
from jax.experimental import pallas as pl
from jax.experimental.pallas import tpu as pltpu
from jax.experimental.pallas import tpu_sc as plsc

B = 16
L = 8192
V = 4
PAD = 5000
LOUT = L + 2 * PAD
HALF = L // 2
ZROW = 5120
LANES = 16


def _zero_template(zpad_v):
    zeros16 = jnp.zeros((LANES,), jnp.float32)

    def body(i, c):
        base = i * (8 * LANES)
        for u in range(8):
            zpad_v[pl.ds(base + u * LANES, LANES)] = zeros16
        return c

    lax.fori_loop(0, ZROW // (8 * LANES), body, 0)


def _compute_half(ids_v, mask_v, data_v):
    def body(i, c):
        base = i * (8 * LANES)
        for u in range(8):
            off = base + u * LANES
            ids16 = ids_v[pl.ds(off, LANES)]
            m16 = mask_v[pl.ds(off, LANES)]
            for v in range(V):
                data_v[pl.ds(v * HALF + off, LANES)] = jnp.where(
                    ids16 == v, m16, 0.0
                )
        return c

    lax.fori_loop(0, HALF // (8 * LANES), body, 0)


@functools.partial(
    pl.kernel,
    mesh=plsc.VectorSubcoreMesh(core_axis_name="c", subcore_axis_name="s"),
    out_type=jax.ShapeDtypeStruct((B * V * LOUT,), jnp.float32),
    scratch_types=[
        pltpu.VMEM((HALF,), jnp.int32),
        pltpu.VMEM((HALF,), jnp.float32),
        pltpu.VMEM((V * HALF,), jnp.float32),
        pltpu.VMEM((ZROW,), jnp.float32),
        pltpu.SemaphoreType.DMA,
        pltpu.SemaphoreType.DMA,
    ],
)
def _one_hot_pad_sc(
    ids_hbm, mask_hbm, out_hbm, ids_v, mask_v, data_v, zpad_v, in_sem, out_sem
):
    info = plsc.get_sparse_core_info()
    wid = lax.axis_index("s") * info.num_cores + lax.axis_index("c")
    b = wid // 2
    h = wid % 2

    in_off = pl.multiple_of(b * L + h * HALF, HALF)
    ids_cp = pltpu.async_copy(ids_hbm.at[pl.ds(in_off, HALF)], ids_v, in_sem)
    mask_cp = pltpu.async_copy(
        mask_hbm.at[pl.ds(in_off, HALF)], mask_v, in_sem
    )

    _zero_template(zpad_v)

    pad_cps = []
    for v in range(V):
        row_off = (b * V + v) * LOUT
        pad_off = pl.multiple_of(row_off + h * (PAD + L), 8)
        pad_cps.append(
            pltpu.async_copy(
                zpad_v.at[pl.ds(0, PAD)], out_hbm.at[pl.ds(pad_off, PAD)],
                out_sem,
            )
        )

    ids_cp.wait()
    mask_cp.wait()
    _compute_half(ids_v, mask_v, data_v)

    data_cps = []
    for v in range(V):
        row_off = (b * V + v) * LOUT
        data_off = pl.multiple_of(row_off + PAD + h * HALF, 8)
        data_cps.append(
            pltpu.async_copy(
                data_v.at[pl.ds(v * HALF, HALF)],
                out_hbm.at[pl.ds(data_off, HALF)],
                out_sem,
            )
        )

    for cp in pad_cps:
        cp.wait()
    for cp in data_cps:
        cp.wait()


def kernel(input_ids, attention_mask):
    ids = input_ids.astype(jnp.int32).reshape(B * L)
    mask = attention_mask.astype(jnp.float32).reshape(B * L)
    out_flat = _one_hot_pad_sc(ids, mask)
    return out_flat.reshape(B, V, LOUT)


# speedup vs baseline: 2.2924x; 2.1028x over previous
import jax
import jax.numpy as jnp
from jax import lax
from jax.experimental import pallas as pl

B = 16
L = 8192
V = 4
PAD = 5000
LOUT = L + 2 * PAD


def _body(ids_ref, mask_ref, out_ref):
    ids = ids_ref[0]
    mask = mask_ref[0]
    ids4 = jnp.broadcast_to(ids, (V, L))
    mask4 = jnp.broadcast_to(mask, (V, L))
    vio = lax.broadcasted_iota(jnp.int32, (V, L), 0)
    center = jnp.where(ids4 == vio, mask4, 0.0)
    out_ref[0] = jnp.zeros((V, LOUT), jnp.float32)
    out_ref[0, :, pl.ds(PAD, L)] = center


def kernel(input_ids, attention_mask):
    ids = input_ids.astype(jnp.int32).reshape(B, 1, L)
    mask = attention_mask.astype(jnp.float32).reshape(B, 1, L)
    return pl.pallas_call(
        _body,
        grid=(B,),
        in_specs=[
            pl.BlockSpec((1, 1, L), lambda b: (b, 0, 0)),
            pl.BlockSpec((1, 1, L), lambda b: (b, 0, 0)),
        ],
        out_specs=pl.BlockSpec((1, V, LOUT), lambda b: (b, 0, 0)),
        out_shape=jax.ShapeDtypeStruct((B, V, LOUT), jnp.float32),
    )(ids, mask)


# speedup vs baseline: 4.1379x; 1.8051x over previous
import jax
import jax.numpy as jnp
from jax import lax
from jax.experimental import pallas as pl

B = 16
L = 8192
V = 4
PAD = 5000
LOUT = L + 2 * PAD


def _body(ids_ref, mask_ref, out_ref):
    b = pl.program_id(0)
    ids = ids_ref[b // 8, b % 8]
    mask = mask_ref[b // 8, b % 8]
    ids4 = jnp.broadcast_to(ids, (V, L))
    mask4 = jnp.broadcast_to(mask, (V, L))
    vio = lax.broadcasted_iota(jnp.int32, (V, L), 0)
    center = jnp.where(ids4 == vio, mask4, 0.0)
    out_ref[0] = jnp.zeros((V, LOUT), jnp.float32)
    out_ref[0, :, pl.ds(PAD, L)] = center


def kernel(input_ids, attention_mask):
    ids = input_ids.astype(jnp.int32).reshape(2, 8, L)
    mask = attention_mask.astype(jnp.float32).reshape(2, 8, L)
    return pl.pallas_call(
        _body,
        grid=(B,),
        in_specs=[
            pl.BlockSpec((2, 8, L), lambda b: (0, 0, 0)),
            pl.BlockSpec((2, 8, L), lambda b: (0, 0, 0)),
        ],
        out_specs=pl.BlockSpec((1, V, LOUT), lambda b: (b, 0, 0)),
        out_shape=jax.ShapeDtypeStruct((B, V, LOUT), jnp.float32),
    )(ids, mask)


# speedup vs baseline: 6.0137x; 1.4533x over previous
import jax
import jax.numpy as jnp
from jax import lax
from jax.experimental import pallas as pl

B = 16
L = 8192
V = 4
PAD = 5000
LOUT = L + 2 * PAD


ROWS = 2


def _body(ids_ref, mask_ref, out_ref):
    g = pl.program_id(0)
    vio = lax.broadcasted_iota(jnp.int32, (V, L), 0)
    for r in range(ROWS):
        b = g * ROWS + r
        ids = ids_ref[b // 8, b % 8]
        mask = mask_ref[b // 8, b % 8]
        ids4 = jnp.broadcast_to(ids, (V, L))
        mask4 = jnp.broadcast_to(mask, (V, L))
        center = jnp.where(ids4 == vio, mask4, 0.0)
        out_ref[r] = jnp.zeros((V, LOUT), jnp.float32)
        out_ref[r, :, pl.ds(PAD, L)] = center


def kernel(input_ids, attention_mask):
    ids = input_ids.astype(jnp.int32).reshape(2, 8, L)
    mask = attention_mask.astype(jnp.float32).reshape(2, 8, L)
    return pl.pallas_call(
        _body,
        grid=(B // ROWS,),
        in_specs=[
            pl.BlockSpec((2, 8, L), lambda b: (0, 0, 0)),
            pl.BlockSpec((2, 8, L), lambda b: (0, 0, 0)),
        ],
        out_specs=pl.BlockSpec((ROWS, V, LOUT), lambda b: (b, 0, 0)),
        out_shape=jax.ShapeDtypeStruct((B, V, LOUT), jnp.float32),
    )(ids, mask)


# speedup vs baseline: 7.5219x; 1.2508x over previous
import jax
import jax.numpy as jnp
from jax import lax
from jax.experimental import pallas as pl

B = 16
L = 8192
V = 4
PAD = 5000
LOUT = L + 2 * PAD


ROWS = 4


def _body(ids_ref, mask_ref, out_ref):
    g = pl.program_id(0)
    vio = lax.broadcasted_iota(jnp.int32, (V, L), 0)
    for r in range(ROWS):
        b = g * ROWS + r
        ids = ids_ref[b // 8, b % 8]
        mask = mask_ref[b // 8, b % 8]
        ids4 = jnp.broadcast_to(ids, (V, L))
        mask4 = jnp.broadcast_to(mask, (V, L))
        center = jnp.where(ids4 == vio, mask4, 0.0)
        out_ref[r] = jnp.zeros((V, LOUT), jnp.float32)
        out_ref[r, :, pl.ds(PAD, L)] = center


def kernel(input_ids, attention_mask):
    ids = input_ids.astype(jnp.int32).reshape(2, 8, L)
    mask = attention_mask.astype(jnp.float32).reshape(2, 8, L)
    return pl.pallas_call(
        _body,
        grid=(B // ROWS,),
        in_specs=[
            pl.BlockSpec((2, 8, L), lambda b: (0, 0, 0)),
            pl.BlockSpec((2, 8, L), lambda b: (0, 0, 0)),
        ],
        out_specs=pl.BlockSpec((ROWS, V, LOUT), lambda b: (b, 0, 0)),
        out_shape=jax.ShapeDtypeStruct((B, V, LOUT), jnp.float32),
    )(ids, mask)


# speedup vs baseline: 8.1704x; 1.0862x over previous
import jax
import jax.numpy as jnp
from jax import lax
from jax.experimental import pallas as pl

B = 16
L = 8192
V = 4
PAD = 5000
LOUT = L + 2 * PAD


ROWS = 8


def _body(ids_ref, mask_ref, out_ref):
    g = pl.program_id(0)
    vio = lax.broadcasted_iota(jnp.int32, (V, L), 0)
    for r in range(ROWS):
        b = g * ROWS + r
        ids = ids_ref[b // 8, b % 8]
        mask = mask_ref[b // 8, b % 8]
        ids4 = jnp.broadcast_to(ids, (V, L))
        mask4 = jnp.broadcast_to(mask, (V, L))
        center = jnp.where(ids4 == vio, mask4, 0.0)
        out_ref[r] = jnp.zeros((V, LOUT), jnp.float32)
        out_ref[r, :, pl.ds(PAD, L)] = center


def kernel(input_ids, attention_mask):
    ids = input_ids.astype(jnp.int32).reshape(2, 8, L)
    mask = attention_mask.astype(jnp.float32).reshape(2, 8, L)
    return pl.pallas_call(
        _body,
        grid=(B // ROWS,),
        in_specs=[
            pl.BlockSpec((2, 8, L), lambda b: (0, 0, 0)),
            pl.BlockSpec((2, 8, L), lambda b: (0, 0, 0)),
        ],
        out_specs=pl.BlockSpec((ROWS, V, LOUT), lambda b: (b, 0, 0)),
        out_shape=jax.ShapeDtypeStruct((B, V, LOUT), jnp.float32),
    )(ids, mask)
